# Initial kernel scaffold; baseline (speedup 1.0000x reference)
import jax
import jax.numpy as jnp
from jax.experimental import pallas as pl


def kernel(x, edge_index, W1, b1, W2, b2):
    raise NotImplementedError("write your pallas kernel here")


# speedup vs baseline: 22.1527x; 22.1527x over previous
import functools

import jax
import jax.numpy as jnp
from jax import lax
from jax.experimental import pallas as pl
from jax.experimental.pallas import tpu as pltpu
from jax.experimental.pallas import tpu_sc as plsc

DH = 16
CHUNK = 128
NW = 32
NSUB = 16


def _sc_mesh():
    return plsc.VectorSubcoreMesh(core_axis_name="c", subcore_axis_name="s")


def _row_scatter(n_nodes, n_chunks_per_tile):
    npad = ((n_nodes + 1 + NSUB * 8 - 1) // (NSUB * 8)) * (NSUB * 8)
    rows_per_sub = npad // NSUB

    @functools.partial(
        pl.kernel,
        out_type=jax.ShapeDtypeStruct((2, npad, DH), jnp.float32),
        mesh=_sc_mesh(),
        compiler_params=pltpu.CompilerParams(use_tc_tiling_on_sc=False),
        scratch_types=[
            pltpu.VMEM((n_chunks_per_tile, CHUNK), jnp.int32),
            pltpu.VMEM((n_chunks_per_tile, CHUNK), jnp.int32),
            pltpu.VMEM((CHUNK, DH), jnp.float32),
            pltpu.VMEM((rows_per_sub, DH), jnp.float32),
            pltpu.VMEM_SHARED((npad, DH), jnp.float32),
            pltpu.SemaphoreType.DMA,
        ],
    )
    def scat(hs, src3, dst3, out, sidx, didx, rows, zbuf, accum, sem):
        c = lax.axis_index("c")
        s = lax.axis_index("s")
        wid = s * 2 + c

        def zero_row(i, _):
            zbuf[i, :] = jnp.zeros((DH,), jnp.float32)
            return 0

        lax.fori_loop(0, rows_per_sub, zero_row, 0)
        pltpu.sync_copy(zbuf, accum.at[pl.ds(s * rows_per_sub, rows_per_sub)])
        pltpu.sync_copy(src3.at[wid], sidx)
        pltpu.sync_copy(dst3.at[wid], didx)
        plsc.subcore_barrier()

        def chunk(j, _):
            pltpu.async_copy(hs.at[sidx.at[j]], rows, sem).wait()
            pltpu.sync_copy(rows, accum.at[didx.at[j]], add=True)
            return 0

        lax.fori_loop(0, n_chunks_per_tile, chunk, 0)
        plsc.subcore_barrier()
        pltpu.sync_copy(
            accum.at[pl.ds(s * rows_per_sub, rows_per_sub)],
            out.at[c, pl.ds(s * rows_per_sub, rows_per_sub)],
        )

    return scat


def _ones_scatter(n_nodes, n_chunks_per_tile):
    npad = ((n_nodes + 1 + NSUB * 8 - 1) // (NSUB * 8)) * (NSUB * 8)
    rows_per_sub = npad // NSUB

    @functools.partial(
        pl.kernel,
        out_type=jax.ShapeDtypeStruct((2, npad, DH), jnp.float32),
        mesh=_sc_mesh(),
        compiler_params=pltpu.CompilerParams(use_tc_tiling_on_sc=False),
        scratch_types=[
            pltpu.VMEM((n_chunks_per_tile, CHUNK), jnp.int32),
            pltpu.VMEM((CHUNK, DH), jnp.float32),
            pltpu.VMEM((rows_per_sub, DH), jnp.float32),
            pltpu.VMEM_SHARED((npad, DH), jnp.float32),
        ],
    )
    def deg(dst3, out, didx, ones, zbuf, accum):
        c = lax.axis_index("c")
        s = lax.axis_index("s")
        wid = s * 2 + c

        def fill(i, _):
            zbuf[i, :] = jnp.zeros((DH,), jnp.float32)
            return 0

        lax.fori_loop(0, rows_per_sub, fill, 0)

        def fill1(i, _):
            ones[i, :] = jnp.ones((DH,), jnp.float32)
            return 0

        lax.fori_loop(0, CHUNK, fill1, 0)
        pltpu.sync_copy(zbuf, accum.at[pl.ds(s * rows_per_sub, rows_per_sub)])
        pltpu.sync_copy(dst3.at[wid], didx)
        plsc.subcore_barrier()

        def chunk(j, _):
            pltpu.sync_copy(ones, accum.at[didx.at[j]], add=True)
            return 0

        lax.fori_loop(0, n_chunks_per_tile, chunk, 0)
        plsc.subcore_barrier()
        pltpu.sync_copy(
            accum.at[pl.ds(s * rows_per_sub, rows_per_sub)],
            out.at[c, pl.ds(s * rows_per_sub, rows_per_sub)],
        )

    return deg


def _tc_in(x, W1, degp, blk):
    n, d_in = x.shape

    def body(x_ref, w_ref, deg_ref, hs_ref, dr_ref):
        dinv = lax.rsqrt(deg_ref[0] + deg_ref[1] + 1.0)
        h = jnp.dot(x_ref[...], w_ref[...], preferred_element_type=jnp.float32)
        hs_ref[...] = h * dinv
        dr_ref[...] = dinv

    return pl.pallas_call(
        body,
        grid=(n // blk,),
        in_specs=[
            pl.BlockSpec((blk, d_in), lambda i: (i, 0)),
            pl.BlockSpec((d_in, DH), lambda i: (0, 0)),
            pl.BlockSpec((2, blk, DH), lambda i: (0, i, 0)),
        ],
        out_specs=[
            pl.BlockSpec((blk, DH), lambda i: (i, 0)),
            pl.BlockSpec((blk, DH), lambda i: (i, 0)),
        ],
        out_shape=[
            jax.ShapeDtypeStruct((n, DH), jnp.float32),
            jax.ShapeDtypeStruct((n, DH), jnp.float32),
        ],
    )(x, W1, degp)


def _tc_mid(s1, hs1, dinvrow, b1, blk):
    n = hs1.shape[0]

    def body(s_ref, h_ref, d_ref, b_ref, o_ref):
        agg = d_ref[...] * (s_ref[0] + s_ref[1] + h_ref[...])
        z = jnp.maximum(agg + b_ref[...], 0.0)
        o_ref[...] = d_ref[...] * z

    return pl.pallas_call(
        body,
        grid=(n // blk,),
        in_specs=[
            pl.BlockSpec((2, blk, DH), lambda i: (0, i, 0)),
            pl.BlockSpec((blk, DH), lambda i: (i, 0)),
            pl.BlockSpec((blk, DH), lambda i: (i, 0)),
            pl.BlockSpec((1, DH), lambda i: (0, 0)),
        ],
        out_specs=pl.BlockSpec((blk, DH), lambda i: (i, 0)),
        out_shape=jax.ShapeDtypeStruct((n, DH), jnp.float32),
    )(s1, hs1, dinvrow, b1)


def _tc_out(s2, hs2, dinvrow, W2, b2, blk):
    n = hs2.shape[0]
    d_out = W2.shape[1]

    def body(s_ref, h_ref, d_ref, w_ref, b_ref, o_ref):
        agg = d_ref[...] * (s_ref[0] + s_ref[1] + h_ref[...])
        y = jnp.dot(agg, w_ref[...], preferred_element_type=jnp.float32)
        y = y + b_ref[...]
        m = jnp.max(y, axis=1, keepdims=True)
        lse = m + jnp.log(jnp.sum(jnp.exp(y - m), axis=1, keepdims=True))
        o_ref[...] = y - lse

    return pl.pallas_call(
        body,
        grid=(n // blk,),
        in_specs=[
            pl.BlockSpec((2, blk, DH), lambda i: (0, i, 0)),
            pl.BlockSpec((blk, DH), lambda i: (i, 0)),
            pl.BlockSpec((blk, DH), lambda i: (i, 0)),
            pl.BlockSpec((DH, d_out), lambda i: (0, 0)),
            pl.BlockSpec((1, d_out), lambda i: (0, 0)),
        ],
        out_specs=pl.BlockSpec((blk, d_out), lambda i: (i, 0)),
        out_shape=jax.ShapeDtypeStruct((n, d_out), jnp.float32),
    )(s2, hs2, dinvrow, W2, b2)


def kernel(x, edge_index, W1, b1, W2, b2):
    n, _ = x.shape
    e = edge_index.shape[1]
    blk = 1000 if n % 1000 == 0 else 8

    epad = ((e + NW * CHUNK - 1) // (NW * CHUNK)) * (NW * CHUNK)
    n_chunks_per_tile = epad // (NW * CHUNK)
    src = edge_index[0].astype(jnp.int32)
    dst = edge_index[1].astype(jnp.int32)
    src3 = jnp.concatenate([src, jnp.zeros((epad - e,), jnp.int32)]).reshape(
        NW, n_chunks_per_tile, CHUNK
    )
    dst3 = jnp.concatenate([dst, jnp.full((epad - e,), n, jnp.int32)]).reshape(
        NW, n_chunks_per_tile, CHUNK
    )

    degp = _ones_scatter(n, n_chunks_per_tile)(dst3)
    hs1, dinvrow = _tc_in(x, W1, degp[:, :n], blk)
    s1 = _row_scatter(n, n_chunks_per_tile)(hs1, src3, dst3)
    hs2 = _tc_mid(s1[:, :n], hs1, dinvrow, b1.reshape(1, DH), blk)
    s2 = _row_scatter(n, n_chunks_per_tile)(hs2, src3, dst3)
    return _tc_out(s2[:, :n], hs2, dinvrow, W2, b2.reshape(1, -1), blk)
